# baseline (speedup 1.0000x reference)
import functools

import jax
import jax.numpy as jnp
from jax import lax
from jax.experimental import pallas as pl
from jax.experimental.pallas import tpu as pltpu
from jax.experimental.pallas import tpu_sc as plsc

_N = 10000
_E = 320000
_D = 128
_H = 128
_P = 64
_NPAD = 10240
_CHUNK = 128
_NCHUNK = _E // _CHUNK
_NW = 32
_RPT = _NPAD // 16
_ROWBLK = 1024
_GRID = _NPAD // _ROWBLK

_C = 80
_NC = _E // (_NW * _C)
_NS = 4

_mesh = plsc.VectorSubcoreMesh(core_axis_name="c", subcore_axis_name="s")


def _zero16():
    return jnp.zeros((16,), jnp.float32)


@functools.partial(
    pl.kernel,
    out_type=jax.ShapeDtypeStruct((2 * _NPAD,), jnp.float32),
    mesh=_mesh,
    scratch_types=[
        pltpu.VMEM((_NS, _C), jnp.int32),
        pltpu.VMEM((_C,), jnp.float32),
        pltpu.VMEM((_RPT,), jnp.float32),
        pltpu.VMEM_SHARED((_NPAD,), jnp.float32),
        pltpu.SemaphoreType.DMA,
        pltpu.SemaphoreType.DMA,
    ],
)
def _sc_degree(ei_hbm, out_hbm, didx, ones_v, zrow, cnt, isem, ssem):
    cid = lax.axis_index("c")
    sid = lax.axis_index("s")
    wid = sid * 2 + cid
    one16 = jnp.ones((16,), jnp.float32)
    z16 = _zero16()
    for j in range(_C // 16):
        ones_v[pl.ds(j * 16, 16)] = one16
    for j in range(_RPT // 16):
        zrow[pl.ds(j * 16, 16)] = z16
    pltpu.sync_copy(zrow, cnt.at[pl.ds(sid * _RPT, _RPT)])
    plsc.subcore_barrier()

    e0 = wid * (_NC * _C)

    def load_idx(g):
        pltpu.async_copy(ei_hbm.at[pl.ds(_E + e0 + g * _C, _C)],
                         didx.at[lax.rem(g, _NS)], isem)

    load_idx(0)
    load_idx(1)

    def body(g, carry):
        p = lax.rem(g, _NS)

        @pl.when(g >= 2)
        def _():
            pltpu.make_async_copy(out_hbm.at[pl.ds(0, _C)], ones_v, ssem).wait()

        @pl.when(g + 2 < _NC)
        def _():
            load_idx(g + 2)

        pltpu.make_async_copy(ei_hbm.at[pl.ds(0, _C)], didx.at[p], isem).wait()
        pltpu.async_copy(ones_v, cnt.at[didx.at[p]], ssem, add=True)
        return carry

    lax.fori_loop(0, _NC, body, 0)
    pltpu.make_async_copy(out_hbm.at[pl.ds(0, _C)], ones_v, ssem).wait()
    pltpu.make_async_copy(out_hbm.at[pl.ds(0, _C)], ones_v, ssem).wait()
    plsc.subcore_barrier()
    pltpu.sync_copy(cnt.at[pl.ds(sid * _RPT, _RPT)],
                    out_hbm.at[pl.ds(cid * _NPAD + sid * _RPT, _RPT)])


@functools.partial(
    pl.kernel,
    out_type=jax.ShapeDtypeStruct((2 * _NPAD, _H), jnp.float32),
    mesh=_mesh,
    scratch_types=[
        pltpu.VMEM((_NS, _C), jnp.int32),
        pltpu.VMEM((_NS, _C), jnp.int32),
        pltpu.VMEM((_NS, _C, _H), jnp.float32),
        pltpu.VMEM((8, _H), jnp.float32),
        pltpu.VMEM_SHARED((_NPAD, _H), jnp.float32),
        pltpu.SemaphoreType.DMA,
        pltpu.SemaphoreType.DMA,
        pltpu.SemaphoreType.DMA,
    ],
)
def _sc_aggregate(ei_hbm, y_hbm, out_hbm, sidx, didx, rows, zbuf,
                  acc, isem, gsem, ssem):
    cid = lax.axis_index("c")
    sid = lax.axis_index("s")
    wid = sid * 2 + cid
    z16 = _zero16()
    for r in range(8):
        for j in range(_H // 16):
            zbuf[r, pl.ds(j * 16, 16)] = z16

    def zbody(i, carry):
        pltpu.sync_copy(zbuf, acc.at[pl.ds(sid * _RPT + i * 8, 8)])
        return carry

    lax.fori_loop(0, _RPT // 8, zbody, 0)
    plsc.subcore_barrier()

    e0 = wid * (_NC * _C)

    def load_idx(g):
        p = lax.rem(g, _NS)
        base = e0 + g * _C
        pltpu.async_copy(ei_hbm.at[pl.ds(base, _C)], sidx.at[p], isem)
        pltpu.async_copy(ei_hbm.at[pl.ds(_E + base, _C)], didx.at[p], isem)

    def drain_idx(p):
        pltpu.make_async_copy(ei_hbm.at[pl.ds(0, _C)], sidx.at[p], isem).wait()
        pltpu.make_async_copy(ei_hbm.at[pl.ds(0, _C)], didx.at[p], isem).wait()

    def fire_gather(g):
        p = lax.rem(g, _NS)
        pltpu.async_copy(y_hbm.at[sidx.at[p]], rows.at[p], gsem)

    def drain_rows(p, sem):
        pltpu.make_async_copy(y_hbm.at[pl.ds(0, _C)], rows.at[p], sem).wait()

    load_idx(0)
    drain_idx(0)
    fire_gather(0)
    load_idx(1)

    def body(g, carry):
        p = lax.rem(g, _NS)

        @pl.when(g >= 2)
        def _():
            drain_rows(lax.rem(g + 2, _NS), ssem)

        @pl.when(g + 2 < _NC)
        def _():
            load_idx(g + 2)

        @pl.when(g + 1 < _NC)
        def _():
            drain_idx(lax.rem(g + 1, _NS))
            fire_gather(g + 1)

        drain_rows(p, gsem)
        pltpu.async_copy(rows.at[p], acc.at[didx.at[p]], ssem, add=True)
        return carry

    lax.fori_loop(0, _NC, body, 0)
    drain_rows(lax.rem(_NC - 2, _NS), ssem)
    drain_rows(lax.rem(_NC - 1, _NS), ssem)
    plsc.subcore_barrier()
    pltpu.sync_copy(acc.at[pl.ds(sid * _RPT, _RPT)],
                    out_hbm.at[pl.ds(cid * _NPAD + sid * _RPT, _RPT)])


def _full(shape):
    return pl.BlockSpec(shape, lambda i: (0,) * len(shape))


def _rowspec(w=_H):
    return pl.BlockSpec((_ROWBLK, w), lambda i: (i, 0))


def _dinv_bcast(cnt):
    d2 = lax.rsqrt(cnt[0] + cnt[1] + 1.0)
    dt = d2.T
    return jnp.concatenate(
        [jnp.broadcast_to(dt[:, a:a + 1], (128, 128)) for a in range(8)],
        axis=0)


def _cntspec():
    return pl.BlockSpec((2, _ROWBLK // 128, 128), lambda i: (0, i, 0))


def _prep_body(x_ref, wm_ref, bm_ref, g_ref, be_ref, mu_ref, va_ref,
               w1_ref, hmlp_ref, u1_ref):
    xb = x_ref[...]
    h = jnp.dot(xb, wm_ref[...], preferred_element_type=jnp.float32) + bm_ref[...]
    h = (h - mu_ref[...]) / jnp.sqrt(va_ref[...] + 1e-5) * g_ref[...] + be_ref[...]
    hmlp_ref[...] = jnp.maximum(h, 0.0)
    u1_ref[...] = jnp.dot(xb, w1_ref[...], preferred_element_type=jnp.float32)


def _tc_prep(x, W_mlp, b_mlp, gam, bet, mu, var, W1):
    return pl.pallas_call(
        _prep_body,
        grid=(_GRID,),
        in_specs=[
            pl.BlockSpec((_ROWBLK, _D), lambda i: (i, 0)),
            _full((_D, _H)), _full((1, _H)), _full((1, _H)), _full((1, _H)),
            _full((1, _H)), _full((1, _H)), _full((_D, _H)),
        ],
        out_specs=[_rowspec(), _rowspec()],
        out_shape=[
            jax.ShapeDtypeStruct((_NPAD, _H), jnp.float32),
            jax.ShapeDtypeStruct((_NPAD, _H), jnp.float32),
        ],
    )(x, W_mlp, b_mlp, gam, bet, mu, var, W1)


def _scale_body(cnt_ref, u1_ref, y1_ref):
    y1_ref[...] = _dinv_bcast(cnt_ref[...]) * u1_ref[...]


def _tc_scale(cnt3, u1):
    return pl.pallas_call(
        _scale_body,
        grid=(_GRID,),
        in_specs=[_cntspec(), _rowspec()],
        out_specs=_rowspec(),
        out_shape=jax.ShapeDtypeStruct((_NPAD, _H), jnp.float32),
    )(cnt3, u1)


def _mid_body(part_ref, y_ref, cnt_ref, b_ref, wn_ref, ynext_ref):
    dinv = _dinv_bcast(cnt_ref[...])
    s = part_ref[0] + part_ref[1] + y_ref[...]
    h = jnp.maximum(dinv * s + b_ref[...], 0.0)
    ynext_ref[...] = dinv * jnp.dot(
        h, wn_ref[...], preferred_element_type=jnp.float32)


def _tc_mid(part, y, cnt3, b, Wn):
    return pl.pallas_call(
        _mid_body,
        grid=(_GRID,),
        in_specs=[
            pl.BlockSpec((2, _ROWBLK, _H), lambda i: (0, i, 0)),
            _rowspec(), _cntspec(),
            _full((1, _H)), _full((_H, _H)),
        ],
        out_specs=_rowspec(),
        out_shape=jax.ShapeDtypeStruct((_NPAD, _H), jnp.float32),
    )(part, y, cnt3, b, Wn)


def _final_body(part_ref, y_ref, cnt_ref, b_ref, hmlp_ref, wg_ref,
                bg_ref, wp1_ref, bp1_ref, wp2_ref, bp2_ref, wc_ref, bc_ref,
                zp_ref, lo_ref):
    s = part_ref[0] + part_ref[1] + y_ref[...]
    hg = jnp.maximum(_dinv_bcast(cnt_ref[...]) * s + b_ref[...], 0.0)
    hmlp = hmlp_ref[...]
    gl = (jnp.dot(hmlp, wg_ref[...][:_H], preferred_element_type=jnp.float32,
                  precision=lax.Precision.HIGHEST)
          + jnp.dot(hg, wg_ref[...][_H:], preferred_element_type=jnp.float32,
                    precision=lax.Precision.HIGHEST)
          + bg_ref[...])
    gate = jax.nn.sigmoid(gl)
    hf = gate * hg + (1.0 - gate) * hmlp
    z1 = jnp.maximum(
        jnp.dot(hf, wp1_ref[...], preferred_element_type=jnp.float32)
        + bp1_ref[...], 0.0)
    zp_ref[...] = jnp.dot(z1, wp2_ref[...],
                          preferred_element_type=jnp.float32) + bp2_ref[...]
    lo_ref[...] = jnp.sum(hf * wc_ref[...], axis=1) + bc_ref[0, 0]


def _tc_final(part, y, cnt3, b, hmlp, Wgb, bg, Wp1, bp1, Wp2, bp2, wc_row, bc):
    return pl.pallas_call(
        _final_body,
        grid=(_GRID,),
        in_specs=[
            pl.BlockSpec((2, _ROWBLK, _H), lambda i: (0, i, 0)),
            _rowspec(), _cntspec(),
            _full((1, _H)),
            _rowspec(),
            _full((2 * _H, 128)), _full((1, 1)),
            _full((_H, _H)), _full((1, _H)),
            _full((_H, _P)), _full((1, _P)),
            _full((1, _H)), _full((1, 1)),
        ],
        out_specs=[
            pl.BlockSpec((_ROWBLK, _P), lambda i: (i, 0)),
            pl.BlockSpec((_ROWBLK,), lambda i: (i,)),
        ],
        out_shape=[
            jax.ShapeDtypeStruct((_N, _P), jnp.float32),
            jax.ShapeDtypeStruct((_N,), jnp.float32),
        ],
    )(part, y, cnt3, b, hmlp, Wgb, bg, Wp1, bp1, Wp2, bp2, wc_row, bc)


def kernel(x, edge_index, W_mlp, b_mlp, bn_gamma, bn_beta, bn_mean, bn_var,
           gcn_Ws, gcn_bs, Wg, bg, Wp1, bp1, Wp2, bp2, Wc, bc):
    ei = edge_index.reshape(2 * _E)
    row2 = lambda v: v.reshape(1, -1)

    cnt3 = _sc_degree(ei).reshape(2, _NPAD // 128, 128)
    hmlp, u1 = _tc_prep(
        x, W_mlp, row2(b_mlp), row2(bn_gamma), row2(bn_beta),
        row2(bn_mean), row2(bn_var), gcn_Ws[0])
    y1 = _tc_scale(cnt3, u1)

    part1 = _sc_aggregate(ei, y1).reshape(2, _NPAD, _H)
    y2 = _tc_mid(part1, y1, cnt3, row2(gcn_bs[0]), gcn_Ws[1])
    part2 = _sc_aggregate(ei, y2).reshape(2, _NPAD, _H)
    y3 = _tc_mid(part2, y2, cnt3, row2(gcn_bs[1]), gcn_Ws[2])
    part3 = _sc_aggregate(ei, y3).reshape(2, _NPAD, _H)

    Wgb = jnp.broadcast_to(Wg, (2 * _H, 128))
    zp, lo = _tc_final(
        part3, y3, cnt3, row2(gcn_bs[2]), hmlp, Wgb, bg.reshape(1, 1),
        Wp1, row2(bp1), Wp2, row2(bp2), row2(Wc.reshape(-1)),
        bc.reshape(1, 1))
    return lo, zp


# speedup vs baseline: 1.0244x; 1.0244x over previous
import functools

import jax
import jax.numpy as jnp
from jax import lax
from jax.experimental import pallas as pl
from jax.experimental.pallas import tpu as pltpu
from jax.experimental.pallas import tpu_sc as plsc

_N = 10000
_E = 320000
_D = 128
_H = 128
_P = 64
_NPAD = 10240
_CHUNK = 128
_NCHUNK = _E // _CHUNK
_NW = 32
_RPT = _NPAD // 16
_ROWBLK = 2048
_GRID = _NPAD // _ROWBLK

_C = 80
_NC = _E // (_NW * _C)
_NS = 4

_mesh = plsc.VectorSubcoreMesh(core_axis_name="c", subcore_axis_name="s")


def _zero16():
    return jnp.zeros((16,), jnp.float32)


@functools.partial(
    pl.kernel,
    out_type=jax.ShapeDtypeStruct((2 * _NPAD,), jnp.float32),
    mesh=_mesh,
    scratch_types=[
        pltpu.VMEM((_NS, _C), jnp.int32),
        pltpu.VMEM((_C,), jnp.float32),
        pltpu.VMEM((_RPT,), jnp.float32),
        pltpu.VMEM_SHARED((_NPAD,), jnp.float32),
        pltpu.SemaphoreType.DMA,
        pltpu.SemaphoreType.DMA,
    ],
)
def _sc_degree(ei_hbm, out_hbm, didx, ones_v, zrow, cnt, isem, ssem):
    cid = lax.axis_index("c")
    sid = lax.axis_index("s")
    wid = sid * 2 + cid
    one16 = jnp.ones((16,), jnp.float32)
    z16 = _zero16()
    for j in range(_C // 16):
        ones_v[pl.ds(j * 16, 16)] = one16
    for j in range(_RPT // 16):
        zrow[pl.ds(j * 16, 16)] = z16
    pltpu.sync_copy(zrow, cnt.at[pl.ds(sid * _RPT, _RPT)])
    plsc.subcore_barrier()

    e0 = wid * (_NC * _C)

    def load_idx(g):
        pltpu.async_copy(ei_hbm.at[pl.ds(_E + e0 + g * _C, _C)],
                         didx.at[lax.rem(g, _NS)], isem)

    load_idx(0)
    load_idx(1)

    def body(g, carry):
        p = lax.rem(g, _NS)

        @pl.when(g >= 2)
        def _():
            pltpu.make_async_copy(out_hbm.at[pl.ds(0, _C)], ones_v, ssem).wait()

        @pl.when(g + 2 < _NC)
        def _():
            load_idx(g + 2)

        pltpu.make_async_copy(ei_hbm.at[pl.ds(0, _C)], didx.at[p], isem).wait()
        pltpu.async_copy(ones_v, cnt.at[didx.at[p]], ssem, add=True)
        return carry

    lax.fori_loop(0, _NC, body, 0)
    pltpu.make_async_copy(out_hbm.at[pl.ds(0, _C)], ones_v, ssem).wait()
    pltpu.make_async_copy(out_hbm.at[pl.ds(0, _C)], ones_v, ssem).wait()
    plsc.subcore_barrier()
    pltpu.sync_copy(cnt.at[pl.ds(sid * _RPT, _RPT)],
                    out_hbm.at[pl.ds(cid * _NPAD + sid * _RPT, _RPT)])


@functools.partial(
    pl.kernel,
    out_type=jax.ShapeDtypeStruct((2 * _NPAD, _H), jnp.float32),
    mesh=_mesh,
    scratch_types=[
        pltpu.VMEM((_NS, _C), jnp.int32),
        pltpu.VMEM((_NS, _C), jnp.int32),
        pltpu.VMEM((_NS, _C, _H), jnp.float32),
        pltpu.VMEM((8, _H), jnp.float32),
        pltpu.VMEM_SHARED((_NPAD, _H), jnp.float32),
        pltpu.SemaphoreType.DMA,
        pltpu.SemaphoreType.DMA,
        pltpu.SemaphoreType.DMA,
    ],
)
def _sc_aggregate(ei_hbm, y_hbm, out_hbm, sidx, didx, rows, zbuf,
                  acc, isem, gsem, ssem):
    cid = lax.axis_index("c")
    sid = lax.axis_index("s")
    wid = sid * 2 + cid
    z16 = _zero16()
    for r in range(8):
        for j in range(_H // 16):
            zbuf[r, pl.ds(j * 16, 16)] = z16

    def zbody(i, carry):
        pltpu.sync_copy(zbuf, acc.at[pl.ds(sid * _RPT + i * 8, 8)])
        return carry

    lax.fori_loop(0, _RPT // 8, zbody, 0)
    plsc.subcore_barrier()

    e0 = wid * (_NC * _C)

    def load_idx(g):
        p = lax.rem(g, _NS)
        base = e0 + g * _C
        pltpu.async_copy(ei_hbm.at[pl.ds(base, _C)], sidx.at[p], isem)
        pltpu.async_copy(ei_hbm.at[pl.ds(_E + base, _C)], didx.at[p], isem)

    def drain_idx(p):
        pltpu.make_async_copy(ei_hbm.at[pl.ds(0, _C)], sidx.at[p], isem).wait()
        pltpu.make_async_copy(ei_hbm.at[pl.ds(0, _C)], didx.at[p], isem).wait()

    def fire_gather(g):
        p = lax.rem(g, _NS)
        pltpu.async_copy(y_hbm.at[sidx.at[p]], rows.at[p], gsem)

    def drain_rows(p, sem):
        pltpu.make_async_copy(y_hbm.at[pl.ds(0, _C)], rows.at[p], sem).wait()

    load_idx(0)
    drain_idx(0)
    fire_gather(0)
    load_idx(1)

    def body(g, carry):
        p = lax.rem(g, _NS)

        @pl.when(g >= 2)
        def _():
            drain_rows(lax.rem(g + 2, _NS), ssem)

        @pl.when(g + 2 < _NC)
        def _():
            load_idx(g + 2)

        @pl.when(g + 1 < _NC)
        def _():
            drain_idx(lax.rem(g + 1, _NS))
            fire_gather(g + 1)

        drain_rows(p, gsem)
        pltpu.async_copy(rows.at[p], acc.at[didx.at[p]], ssem, add=True)
        return carry

    lax.fori_loop(0, _NC, body, 0)
    drain_rows(lax.rem(_NC - 2, _NS), ssem)
    drain_rows(lax.rem(_NC - 1, _NS), ssem)
    plsc.subcore_barrier()
    pltpu.sync_copy(acc.at[pl.ds(sid * _RPT, _RPT)],
                    out_hbm.at[pl.ds(cid * _NPAD + sid * _RPT, _RPT)])


def _full(shape):
    return pl.BlockSpec(shape, lambda i: (0,) * len(shape))


def _rowspec(w=_H):
    return pl.BlockSpec((_ROWBLK, w), lambda i: (i, 0))


def _dinv_bcast(cnt):
    d2 = lax.rsqrt(cnt[0] + cnt[1] + 1.0)
    dt = d2.T
    return jnp.concatenate(
        [jnp.broadcast_to(dt[:, a:a + 1], (128, 128))
         for a in range(_ROWBLK // 128)],
        axis=0)


def _cntspec():
    return pl.BlockSpec((2, _ROWBLK // 128, 128), lambda i: (0, i, 0))


def _prep_body(x_ref, wm_ref, bm_ref, g_ref, be_ref, mu_ref, va_ref,
               w1_ref, hmlp_ref, u1_ref):
    xb = x_ref[...]
    h = jnp.dot(xb, wm_ref[...], preferred_element_type=jnp.float32) + bm_ref[...]
    h = (h - mu_ref[...]) / jnp.sqrt(va_ref[...] + 1e-5) * g_ref[...] + be_ref[...]
    hmlp_ref[...] = jnp.maximum(h, 0.0)
    u1_ref[...] = jnp.dot(xb, w1_ref[...], preferred_element_type=jnp.float32)


def _tc_prep(x, W_mlp, b_mlp, gam, bet, mu, var, W1):
    return pl.pallas_call(
        _prep_body,
        grid=(_GRID,),
        in_specs=[
            pl.BlockSpec((_ROWBLK, _D), lambda i: (i, 0)),
            _full((_D, _H)), _full((1, _H)), _full((1, _H)), _full((1, _H)),
            _full((1, _H)), _full((1, _H)), _full((_D, _H)),
        ],
        out_specs=[_rowspec(), _rowspec()],
        out_shape=[
            jax.ShapeDtypeStruct((_NPAD, _H), jnp.float32),
            jax.ShapeDtypeStruct((_NPAD, _H), jnp.float32),
        ],
    )(x, W_mlp, b_mlp, gam, bet, mu, var, W1)


def _scale_body(cnt_ref, u1_ref, y1_ref):
    y1_ref[...] = _dinv_bcast(cnt_ref[...]) * u1_ref[...]


def _tc_scale(cnt3, u1):
    return pl.pallas_call(
        _scale_body,
        grid=(_GRID,),
        in_specs=[_cntspec(), _rowspec()],
        out_specs=_rowspec(),
        out_shape=jax.ShapeDtypeStruct((_NPAD, _H), jnp.float32),
    )(cnt3, u1)


def _mid_body(part_ref, y_ref, cnt_ref, b_ref, wn_ref, ynext_ref):
    dinv = _dinv_bcast(cnt_ref[...])
    s = part_ref[0] + part_ref[1] + y_ref[...]
    h = jnp.maximum(dinv * s + b_ref[...], 0.0)
    ynext_ref[...] = dinv * jnp.dot(
        h, wn_ref[...], preferred_element_type=jnp.float32)


def _tc_mid(part, y, cnt3, b, Wn):
    return pl.pallas_call(
        _mid_body,
        grid=(_GRID,),
        in_specs=[
            pl.BlockSpec((2, _ROWBLK, _H), lambda i: (0, i, 0)),
            _rowspec(), _cntspec(),
            _full((1, _H)), _full((_H, _H)),
        ],
        out_specs=_rowspec(),
        out_shape=jax.ShapeDtypeStruct((_NPAD, _H), jnp.float32),
    )(part, y, cnt3, b, Wn)


def _final_body(part_ref, y_ref, cnt_ref, b_ref, hmlp_ref, wg_ref,
                bg_ref, wp1_ref, bp1_ref, wp2_ref, bp2_ref, wc_ref, bc_ref,
                zp_ref, lo_ref):
    s = part_ref[0] + part_ref[1] + y_ref[...]
    hg = jnp.maximum(_dinv_bcast(cnt_ref[...]) * s + b_ref[...], 0.0)
    hmlp = hmlp_ref[...]
    gl = (jnp.dot(hmlp, wg_ref[...][:_H], preferred_element_type=jnp.float32)
          + jnp.dot(hg, wg_ref[...][_H:], preferred_element_type=jnp.float32)
          + bg_ref[...])
    gate = jax.nn.sigmoid(gl)
    hf = gate * hg + (1.0 - gate) * hmlp
    z1 = jnp.maximum(
        jnp.dot(hf, wp1_ref[...], preferred_element_type=jnp.float32)
        + bp1_ref[...], 0.0)
    zp_ref[...] = jnp.dot(z1, wp2_ref[...],
                          preferred_element_type=jnp.float32) + bp2_ref[...]
    lo_ref[...] = jnp.sum(hf * wc_ref[...], axis=1) + bc_ref[0, 0]


def _tc_final(part, y, cnt3, b, hmlp, Wgb, bg, Wp1, bp1, Wp2, bp2, wc_row, bc):
    return pl.pallas_call(
        _final_body,
        grid=(_GRID,),
        in_specs=[
            pl.BlockSpec((2, _ROWBLK, _H), lambda i: (0, i, 0)),
            _rowspec(), _cntspec(),
            _full((1, _H)),
            _rowspec(),
            _full((2 * _H, 128)), _full((1, 1)),
            _full((_H, _H)), _full((1, _H)),
            _full((_H, _P)), _full((1, _P)),
            _full((1, _H)), _full((1, 1)),
        ],
        out_specs=[
            pl.BlockSpec((_ROWBLK, _P), lambda i: (i, 0)),
            pl.BlockSpec((_ROWBLK,), lambda i: (i,)),
        ],
        out_shape=[
            jax.ShapeDtypeStruct((_N, _P), jnp.float32),
            jax.ShapeDtypeStruct((_N,), jnp.float32),
        ],
    )(part, y, cnt3, b, hmlp, Wgb, bg, Wp1, bp1, Wp2, bp2, wc_row, bc)


def kernel(x, edge_index, W_mlp, b_mlp, bn_gamma, bn_beta, bn_mean, bn_var,
           gcn_Ws, gcn_bs, Wg, bg, Wp1, bp1, Wp2, bp2, Wc, bc):
    ei = edge_index.reshape(2 * _E)
    row2 = lambda v: v.reshape(1, -1)

    cnt3 = _sc_degree(ei).reshape(2, _NPAD // 128, 128)
    hmlp, u1 = _tc_prep(
        x, W_mlp, row2(b_mlp), row2(bn_gamma), row2(bn_beta),
        row2(bn_mean), row2(bn_var), gcn_Ws[0])
    y1 = _tc_scale(cnt3, u1)

    part1 = _sc_aggregate(ei, y1).reshape(2, _NPAD, _H)
    y2 = _tc_mid(part1, y1, cnt3, row2(gcn_bs[0]), gcn_Ws[1])
    part2 = _sc_aggregate(ei, y2).reshape(2, _NPAD, _H)
    y3 = _tc_mid(part2, y2, cnt3, row2(gcn_bs[1]), gcn_Ws[2])
    part3 = _sc_aggregate(ei, y3).reshape(2, _NPAD, _H)

    Wgb = jnp.broadcast_to(Wg, (2 * _H, 128))
    zp, lo = _tc_final(
        part3, y3, cnt3, row2(gcn_bs[2]), hmlp, Wgb, bg.reshape(1, 1),
        Wp1, row2(bp1), Wp2, row2(bp2), row2(Wc.reshape(-1)),
        bc.reshape(1, 1))
    return lo, zp
